# baseline (speedup 1.0000x reference)
import functools

import jax
import jax.numpy as jnp
from jax import lax
from jax.experimental import pallas as pl
from jax.experimental.pallas import tpu as pltpu
from jax.experimental.pallas import tpu_sc as plsc

_GAMMA = 0.01
_IG = 100.0

_B, _G, _S, _L = 64, 2048, 64, 4
_NBH = 2
_NGT = 16
_GC = _G // _NGT
_NQ = 2
_SU = 4


def _sc_clause(x_hbm, idx_hbm, outm_hbm, outs_hbm, xloc, iref, outm_v, outs_v):
    wid = lax.axis_index("s") * 2 + lax.axis_index("c")
    bh = wid // _NGT
    gt = wid % _NGT

    pltpu.sync_copy(x_hbm.at[bh], xloc)
    pltpu.sync_copy(idx_hbm.at[gt], iref)

    neg = jnp.full((16,), -1e30, jnp.float32)
    zero = jnp.zeros((16,), jnp.float32)

    def g_body(g, _):
        p0 = g * (_S * _L)

        def s_body(sb, st):
            m0, s0, m1, s1 = st
            iv = iref[pl.ds(p0 + sb * (_SU * _L), _SU * _L)]
            for u in range(_SU):
                i0 = iv[u * _L]
                i1 = iv[u * _L + 1]
                i2 = iv[u * _L + 2]
                i3 = iv[u * _L + 3]
                a0 = xloc[i0, 0] * xloc[i1, 0] * xloc[i2, 0] * xloc[i3, 0]
                a1 = xloc[i0, 1] * xloc[i1, 1] * xloc[i2, 1] * xloc[i3, 1]
                m0n = jnp.maximum(m0, a0)
                s0 = (s0 * jnp.exp((m0 - m0n) * _IG)
                      + jnp.exp((a0 - m0n) * _IG))
                m0 = m0n
                m1n = jnp.maximum(m1, a1)
                s1 = (s1 * jnp.exp((m1 - m1n) * _IG)
                      + jnp.exp((a1 - m1n) * _IG))
                m1 = m1n
            return m0, s0, m1, s1

        m0, s0, m1, s1 = lax.fori_loop(
            0, _S // _SU, s_body, (neg, zero, neg, zero))
        outm_v[g, 0] = m0
        outm_v[g, 1] = m1
        outs_v[g, 0] = s0
        outs_v[g, 1] = s1
        return 0

    lax.fori_loop(0, _GC, g_body, 0)

    pltpu.sync_copy(outm_v, outm_hbm.at[bh, gt])
    pltpu.sync_copy(outs_v, outs_hbm.at[bh, gt])


_sc_call = functools.partial(
    pl.kernel,
    out_type=[
        jax.ShapeDtypeStruct((_NBH, _NGT, _GC, _NQ, 16), jnp.float32),
        jax.ShapeDtypeStruct((_NBH, _NGT, _GC, _NQ, 16), jnp.float32),
    ],
    mesh=plsc.VectorSubcoreMesh(core_axis_name="c", subcore_axis_name="s"),
    compiler_params=pltpu.CompilerParams(
        needs_layout_passes=False, use_tc_tiling_on_sc=False),
    scratch_types=[
        pltpu.VMEM((_G, _NQ, 16), jnp.float32),
        pltpu.VMEM((_GC * _S * _L,), jnp.int32),
        pltpu.VMEM((_GC, _NQ, 16), jnp.float32),
        pltpu.VMEM((_GC, _NQ, 16), jnp.float32),
    ],
)(_sc_clause)


def _fin_body(m_ref, s_ref, o_ref):
    o_ref[...] = m_ref[...] + _GAMMA * jnp.log(s_ref[...])


_finish = pl.pallas_call(
    _fin_body,
    out_shape=jax.ShapeDtypeStruct((_B, _G), jnp.float32),
)


def kernel(x, I):
    xr = x.reshape(_NBH, _NQ, 16, _G).transpose(0, 3, 1, 2)
    idx = I[0].reshape(_NGT, _GC * _S * _L)
    outm5, outs5 = _sc_call(xr, idx)
    m = jnp.transpose(outm5, (0, 3, 4, 1, 2)).reshape(_B, _G)
    sv = jnp.transpose(outs5, (0, 3, 4, 1, 2)).reshape(_B, _G)
    return _finish(m, sv)


# speedup vs baseline: 1.0072x; 1.0072x over previous
import functools

import jax
import jax.numpy as jnp
from jax import lax
from jax.experimental import pallas as pl
from jax.experimental.pallas import tpu as pltpu
from jax.experimental.pallas import tpu_sc as plsc

_GAMMA = 0.01
_IG = 100.0

_B, _G, _S, _L = 64, 2048, 64, 4
_NBH = 2
_NGT = 16
_GC = _G // _NGT
_NQ = 2
_CH = 128
_NCH = _GC * _S * _L // _CH
_NB = 4


def _sc_clause(x_hbm, idx_hbm, outm_hbm, outs_hbm,
               xspm, iref, outm_v, outs_v,
               b0, b1, b2, b3, s0_, s1_, s2_, s3_):
    bh = lax.axis_index("c")
    gt = lax.axis_index("s")
    bufs = (b0, b1, b2, b3)
    sems = (s0_, s1_, s2_, s3_)

    @pl.when(gt == 0)
    def _():
        pltpu.sync_copy(x_hbm.at[bh], xspm)

    pltpu.sync_copy(idx_hbm.at[gt], iref)
    plsc.subcore_barrier()

    for slot in range(_NB):
        pltpu.async_copy(xspm.at[iref.at[slot]], bufs[slot], sems[slot])

    neg = jnp.full((16,), -1e30, jnp.float32)
    zero = jnp.zeros((16,), jnp.float32)

    def compute_chunk(buf, st):
        def u_body(ub, st):
            m0, s0, m1, s1 = st
            for v in range(4):
                r = (ub * 4 + v) * _L
                a0 = buf[r, 0] * buf[r + 1, 0] * buf[r + 2, 0] * buf[r + 3, 0]
                a1 = buf[r, 1] * buf[r + 1, 1] * buf[r + 2, 1] * buf[r + 3, 1]
                m0n = jnp.maximum(m0, a0)
                s0 = (s0 * jnp.exp((m0 - m0n) * _IG)
                      + jnp.exp((a0 - m0n) * _IG))
                m0 = m0n
                m1n = jnp.maximum(m1, a1)
                s1 = (s1 * jnp.exp((m1 - m1n) * _IG)
                      + jnp.exp((a1 - m1n) * _IG))
                m1 = m1n
            return m0, s0, m1, s1

        return lax.fori_loop(0, _CH // (4 * _L), u_body, st)

    def grp_body(cg, _):
        for gg in range(_NB // 2):
            st = (neg, zero, neg, zero)
            for half in range(2):
                slot = gg * 2 + half
                c = cg * _NB + slot
                pltpu.make_async_copy(
                    xspm.at[iref.at[slot]], bufs[slot], sems[slot]).wait()
                st = compute_chunk(bufs[slot], st)

                @pl.when(cg < _NCH // _NB - 1)
                def _(c=c, slot=slot):
                    pltpu.async_copy(
                        xspm.at[iref.at[c + _NB]], bufs[slot], sems[slot])

            g = cg * 2 + gg
            m0, s0, m1, s1 = st
            outm_v[g, 0] = m0
            outm_v[g, 1] = m1
            outs_v[g, 0] = s0
            outs_v[g, 1] = s1
        return 0

    lax.fori_loop(0, _NCH // _NB, grp_body, 0)

    pltpu.sync_copy(outm_v, outm_hbm.at[bh, gt])
    pltpu.sync_copy(outs_v, outs_hbm.at[bh, gt])


_sc_call = functools.partial(
    pl.kernel,
    out_type=[
        jax.ShapeDtypeStruct((_NBH, _NGT, _GC, _NQ, 16), jnp.float32),
        jax.ShapeDtypeStruct((_NBH, _NGT, _GC, _NQ, 16), jnp.float32),
    ],
    mesh=plsc.VectorSubcoreMesh(core_axis_name="c", subcore_axis_name="s"),
    compiler_params=pltpu.CompilerParams(
        needs_layout_passes=False, use_tc_tiling_on_sc=False),
    scratch_types=[
        pltpu.MemorySpace.VMEM_SHARED((_G, _NQ, 16), jnp.float32),
        pltpu.VMEM((_NCH, _CH), jnp.int32),
        pltpu.VMEM((_GC, _NQ, 16), jnp.float32),
        pltpu.VMEM((_GC, _NQ, 16), jnp.float32),
    ]
    + [pltpu.VMEM((_CH, _NQ, 16), jnp.float32)] * _NB
    + [pltpu.SemaphoreType.DMA] * _NB,
)(_sc_clause)


def _fin_body(m_ref, s_ref, o_ref):
    o_ref[...] = m_ref[...] + _GAMMA * jnp.log(s_ref[...])


_finish = pl.pallas_call(
    _fin_body,
    out_shape=jax.ShapeDtypeStruct((_B, _G), jnp.float32),
)


def kernel(x, I):
    xr = x.reshape(_NBH, _NQ, 16, _G).transpose(0, 3, 1, 2)
    idx = I[0].reshape(_NGT, _NCH, _CH)
    outm5, outs5 = _sc_call(xr, idx)
    m = jnp.transpose(outm5, (0, 3, 4, 1, 2)).reshape(_B, _G)
    sv = jnp.transpose(outs5, (0, 3, 4, 1, 2)).reshape(_B, _G)
    return _finish(m, sv)
